# baseline (speedup 1.0000x reference)
import jax
import jax.numpy as jnp
from jax.experimental import pallas as pl

MARGIN_ = 1.0
B_ = 1024
D_ = 128
NCLS_ = 16
MASK_K_ = 100.0
BLK_ = 128


def _tree_sum(parts):
    parts = list(parts)
    while len(parts) > 1:
        half = [a + b for a, b in zip(parts[::2], parts[1::2])]
        if len(parts) % 2:
            half.append(parts[-1])
        parts = half
    return parts[0]


def _loss_kernel(e_ref, t_ref, o_ref):
    e = e_ref[...]
    t = t_ref[...]
    tcol = t.reshape(B_, 1)
    trow = t.reshape(1, B_)
    nsq = jnp.sum(e * e, axis=1, keepdims=True)
    ones_col = jnp.ones((B_, 1), dtype=jnp.float32)

    cls_row = jax.lax.broadcasted_iota(jnp.int32, (1, NCLS_), 1)
    oh_col = (tcol == cls_row).astype(jnp.float32)
    lhs = jnp.concatenate([e, nsq, ones_col, MASK_K_ * oh_col], axis=1)
    rhs = jnp.concatenate([-2.0 * e, ones_col, nsq, oh_col], axis=1)
    nb = B_ // BLK_
    diag_parts = []
    off_parts = []
    for bi in range(nb):
        for bj in range(bi, nb):
            sq_masked = jax.lax.dot_general(
                lhs[bi * BLK_:(bi + 1) * BLK_, :],
                rhs[bj * BLK_:(bj + 1) * BLK_, :],
                (((1,), (1,)), ((), ())),
                preferred_element_type=jnp.float32)
            sq_c = jnp.maximum(sq_masked, 1e-12)
            d = sq_c * jax.lax.rsqrt(sq_c)
            r = jnp.maximum(MARGIN_ - d, 0.0)
            part = jnp.sum(r * r, keepdims=True)
            (diag_parts if bi == bj else off_parts).append(part)
    neg_sum = _tree_sum(diag_parts) + 2.0 * _tree_sum(off_parts)

    cls_col = jax.lax.broadcasted_iota(jnp.int32, (NCLS_, 1), 0)
    oh_row = (trow == cls_col).astype(jnp.float32)
    s = jax.lax.dot_general(
        oh_row, e, (((1,), (0,)), ((), ())),
        preferred_element_type=jnp.float32,
        precision=jax.lax.Precision.HIGHEST)
    cls_nsq = jax.lax.dot_general(
        oh_row, nsq, (((1,), (0,)), ((), ())),
        preferred_element_type=jnp.float32,
        precision=jax.lax.Precision.HIGHEST)
    cnt = jnp.sum(oh_row, axis=1, keepdims=True)
    s_norm = jnp.sum(s * s, axis=1, keepdims=True)
    pos_sum = 2.0 * jnp.sum(cnt * cls_nsq - s_norm, keepdims=True)

    n_pairs = B_ * (B_ - 1) // 2
    o_ref[...] = (pos_sum + neg_sum) * (0.5 / n_pairs)


def kernel(embeddings, target):
    out = pl.pallas_call(
        _loss_kernel,
        out_shape=jax.ShapeDtypeStruct((1, 1), jnp.float32),
    )(embeddings, target)
    return out.reshape(())


# speedup vs baseline: 1.0960x; 1.0960x over previous
import jax
import jax.numpy as jnp
from jax.experimental import pallas as pl

MARGIN_ = 1.0
B_ = 1024
D_ = 128
NCLS_ = 16
MASK_K_ = 100.0
BLK_ = 128


def _tree_sum(parts):
    parts = list(parts)
    while len(parts) > 1:
        half = [a + b for a, b in zip(parts[::2], parts[1::2])]
        if len(parts) % 2:
            half.append(parts[-1])
        parts = half
    return parts[0]


def _loss_kernel(e_ref, t_ref, o_ref):
    e = e_ref[...]
    t = t_ref[...]
    tcol = t.reshape(B_, 1)
    trow = t.reshape(1, B_)
    nsq = jnp.sum(e * e, axis=1, keepdims=True)
    ones_col = jnp.ones((B_, 1), dtype=jnp.float32)

    cls_row = jax.lax.broadcasted_iota(jnp.int32, (1, NCLS_), 1)
    oh_col = (tcol == cls_row).astype(jnp.float32)
    lhs = jnp.concatenate([e, nsq, ones_col, MASK_K_ * oh_col], axis=1)
    rhs = jnp.concatenate([-2.0 * e, ones_col, nsq, oh_col], axis=1)
    nb = B_ // BLK_
    diag_parts = []
    off_parts = []
    for bi in range(nb):
        for bj in range(bi, nb):
            sq_masked = jax.lax.dot_general(
                lhs[bi * BLK_:(bi + 1) * BLK_, :],
                rhs[bj * BLK_:(bj + 1) * BLK_, :],
                (((1,), (1,)), ((), ())),
                preferred_element_type=jnp.float32)
            sq_c = jnp.maximum(sq_masked, 1e-12)
            d = sq_c * jax.lax.rsqrt(sq_c)
            r = jnp.maximum(MARGIN_ - d, 0.0)
            part = jnp.sum(r * r, keepdims=True)
            (diag_parts if bi == bj else off_parts).append(part)
    neg_sum = _tree_sum(diag_parts) + 2.0 * _tree_sum(off_parts)

    cls_col = jax.lax.broadcasted_iota(jnp.int32, (NCLS_, 1), 0)
    oh_row = (trow == cls_col).astype(jnp.float32)
    s = jax.lax.dot_general(
        oh_row, e, (((1,), (0,)), ((), ())),
        preferred_element_type=jnp.float32)
    cls_nsq = jax.lax.dot_general(
        oh_row, nsq, (((1,), (0,)), ((), ())),
        preferred_element_type=jnp.float32)
    cnt = jnp.sum(oh_row, axis=1, keepdims=True)
    s_norm = jnp.sum(s * s, axis=1, keepdims=True)
    pos_sum = 2.0 * jnp.sum(cnt * cls_nsq - s_norm, keepdims=True)

    n_pairs = B_ * (B_ - 1) // 2
    o_ref[...] = (pos_sum + neg_sum) * (0.5 / n_pairs)


def kernel(embeddings, target):
    out = pl.pallas_call(
        _loss_kernel,
        out_shape=jax.ShapeDtypeStruct((1, 1), jnp.float32),
    )(embeddings, target)
    return out.reshape(())
